# Initial kernel scaffold; baseline (speedup 1.0000x reference)
import jax
import jax.numpy as jnp
from jax.experimental import pallas as pl


def kernel(X, neighbors, gru_kernel, gru_rec_kernel, gru_bias, rel_emb, proj_W, proj_b, att_W, att_b, relatt_W, relatt_b, pred_W, pred_b):
    raise NotImplementedError("write your pallas kernel here")


# speedup vs baseline: 3.3252x; 3.3252x over previous
import functools
import jax
import jax.numpy as jnp
from jax import lax
from jax.experimental import pallas as pl
from jax.experimental.pallas import tpu as pltpu
from jax.experimental.pallas import tpu_sc as plsc

B, N, L, FIN = 4, 2000, 50, 8
U, R, K, NLAB, RD = 64, 4, 20, 3, 32
M = B * N
NP = N + 1
P = B * R
WROW = 128
ND = P * N
CH = 16
NCHUNK = ND // CH
NWORK = 32
EPB = CH * K
MB = 1000
GRID_A = M // MB


def _leaky(x):
    return jnp.where(x < 0, 0.2 * x, x)


def _stage_a_body(x_ref, gk_ref, grk_ref, gb_ref, wall_ref, ball_ref,
                  bd_ref, wc_ref, cv_ref, node_ref, proj_ref, sc_ref):
    x = x_ref[...]
    gk = gk_ref[...]
    grk = grk_ref[...]
    b0 = gb_ref[0:1, :]
    rb = gb_ref[1:2, :]
    h = jnp.zeros((MB, U), jnp.float32)
    for t in range(L):
        xt = x[:, t * FIN:(t + 1) * FIN]
        pre = jnp.dot(xt, gk, preferred_element_type=jnp.float32) + b0
        inner = jnp.dot(h, grk, preferred_element_type=jnp.float32) + rb
        z = jax.nn.sigmoid(pre[:, :U] + inner[:, :U])
        r = jax.nn.sigmoid(pre[:, U:2 * U] + inner[:, U:2 * U])
        hh = jnp.tanh(pre[:, 2 * U:] + r * inner[:, 2 * U:])
        h = z * h + (1.0 - z) * hh
    node_ref[...] = h
    pa = _leaky(jnp.dot(h, wall_ref[...], preferred_element_type=jnp.float32)
                + ball_ref[...])
    proj_ref[...] = pa
    sc_ref[...] = (jnp.dot(pa, bd_ref[...], preferred_element_type=jnp.float32)
                   + jnp.dot(h, wc_ref[...], preferred_element_type=jnp.float32)
                   + cv_ref[...])


def _run_stage_a(Xr, gk, grk, gb, Wall, ball, BD, Wc, cvec):
    full = lambda s: pl.BlockSpec(s, lambda i: (0, 0))
    return pl.pallas_call(
        _stage_a_body,
        grid=(GRID_A,),
        in_specs=[
            pl.BlockSpec((MB, L * FIN), lambda i: (i, 0)),
            full((FIN, 3 * U)), full((U, 3 * U)), full((2, 3 * U)),
            full((U, R * U)), full((1, R * U)),
            full((R * U, 8)), full((U, 8)), full((1, 8)),
        ],
        out_specs=[
            pl.BlockSpec((MB, U), lambda i: (i, 0)),
            pl.BlockSpec((MB, R * U), lambda i: (i, 0)),
            pl.BlockSpec((MB, 8), lambda i: (i, 0)),
        ],
        out_shape=[
            jax.ShapeDtypeStruct((M, U), jnp.float32),
            jax.ShapeDtypeStruct((M, R * U), jnp.float32),
            jax.ShapeDtypeStruct((M, 8), jnp.float32),
        ],
    )(Xr, gk, grk, gb, Wall, ball, BD, Wc, cvec)


NCHUNK_PAD = 2016
ND_PAD = NCHUNK_PAD * CH


def _stage_c_kernel(table_hbm, gidx_hbm, ctab_hbm, colmap_hbm, out_hbm,
                    gidx_v, rows_v, cv_v, colmap_v, w_v, acc_v, sem):
    nc = 2
    wid = lax.axis_index("s") * nc + lax.axis_index("c")
    pltpu.sync_copy(colmap_hbm, colmap_v)

    def chunk_body(j, carry):
        c = wid + NWORK * j
        dstart = c * CH
        pltpu.sync_copy(gidx_hbm.at[pl.ds(c * EPB, EPB)], gidx_v)
        pltpu.sync_copy(ctab_hbm.at[pl.ds(dstart, CH)], cv_v)

        cps = [pltpu.async_copy(table_hbm.at[gidx_v.at[pl.ds(q * 80, 80)]],
                                rows_v.at[pl.ds(q * 80, 80)], sem)
               for q in range(4)]
        for cp in cps:
            cp.wait()

        cv = cv_v[...]
        col = jnp.full((16,), U, jnp.int32)
        mx = jnp.full((16,), -1e30, jnp.float32)
        for k in range(K):
            rid = lax.iota(jnp.int32, 16) * K + k
            t = cv + plsc.load_gather(rows_v, [rid, col])
            t = jnp.where(t < 0.0, 0.2 * t, t)
            w_v[pl.ds(k * CH, CH)] = t
            mx = jnp.maximum(mx, t)
        den = jnp.zeros((16,), jnp.float32)
        for k in range(K):
            e = jnp.exp(w_v[pl.ds(k * CH, CH)] - mx)
            w_v[pl.ds(k * CH, CH)] = e
            den = den + e
        inv = 1.0 / den
        for k in range(K):
            w_v[pl.ds(k * CH, CH)] = w_v[pl.ds(k * CH, CH)] * inv

        def f_body(f, carry2):
            colf = colmap_v[pl.ds(f * CH, CH)]
            acc = jnp.zeros((16,), jnp.float32)
            for k in range(K):
                rid = lax.iota(jnp.int32, 16) * K + k
                acc = acc + (w_v[pl.ds(k * CH, CH)]
                             * plsc.load_gather(rows_v, [rid, colf]))
            plsc.store_scatter(acc_v, [lax.iota(jnp.int32, 16), colf], acc)
            return carry2

        lax.fori_loop(0, U, f_body, 0)
        pltpu.sync_copy(acc_v, out_hbm.at[pl.ds(dstart, CH)])
        return carry

    lax.fori_loop(0, NCHUNK_PAD // NWORK, chunk_body, 0)


def _run_stage_c(table, gidx, ctab):
    colmap = jnp.broadcast_to(
        jnp.arange(U, dtype=jnp.int32)[:, None], (U, CH)).reshape(U * CH)
    mesh = plsc.VectorSubcoreMesh(core_axis_name="c", subcore_axis_name="s")
    kfn = functools.partial(
        pl.kernel,
        mesh=mesh,
        compiler_params=pltpu.CompilerParams(needs_layout_passes=False),
        out_type=jax.ShapeDtypeStruct((ND_PAD, U), jnp.float32),
        scratch_types=[
            pltpu.VMEM((EPB,), jnp.int32),
            pltpu.VMEM((EPB, WROW), jnp.float32),
            pltpu.VMEM((CH,), jnp.float32),
            pltpu.VMEM((U * CH,), jnp.int32),
            pltpu.VMEM((K * CH,), jnp.float32),
            pltpu.VMEM((CH, U), jnp.float32),
            pltpu.SemaphoreType.DMA,
        ],
    )(_stage_c_kernel)
    return kfn(table, gidx, ctab, colmap)


def _stage_d_body(node_ref, rep_ref, bd2_ref, wc2_ref, c2_ref,
                  pw_ref, pb_ref, out_ref):
    h = node_ref[...]
    rep = rep_ref[...]
    sc2 = (jnp.dot(rep, bd2_ref[...], preferred_element_type=jnp.float32)
           + jnp.dot(h, wc2_ref[...], preferred_element_type=jnp.float32)
           + c2_ref[...])
    sc2 = _leaky(sc2)
    m2 = jnp.max(sc2, axis=1, keepdims=True)
    e2 = jnp.exp(sc2 - m2)
    w2 = e2 / jnp.sum(e2, axis=1, keepdims=True)
    upd = h
    for r in range(R):
        upd = upd + rep[:, r * U:(r + 1) * U] * w2[:, r:r + 1]
    logits = (jnp.dot(upd, pw_ref[...], preferred_element_type=jnp.float32)
              + pb_ref[...])
    ml = jnp.max(logits, axis=1, keepdims=True)
    el = jnp.exp(logits - ml)
    out_ref[...] = el / jnp.sum(el, axis=1, keepdims=True)


def _run_stage_d(node, rep_all, bd2, wc2, c2, pw, pb):
    full = lambda s: pl.BlockSpec(s, lambda i: (0, 0))
    return pl.pallas_call(
        _stage_d_body,
        grid=(GRID_A,),
        in_specs=[
            pl.BlockSpec((MB, U), lambda i: (i, 0)),
            pl.BlockSpec((MB, R * U), lambda i: (i, 0)),
            full((R * U, R)), full((U, R)), full((1, R)),
            full((U, NLAB)), full((1, NLAB)),
        ],
        out_specs=pl.BlockSpec((MB, NLAB), lambda i: (i, 0)),
        out_shape=jax.ShapeDtypeStruct((M, NLAB), jnp.float32),
    )(node, rep_all, bd2, wc2, c2, pw, pb)


def kernel(X, neighbors, gru_kernel, gru_rec_kernel, gru_bias, rel_emb,
           proj_W, proj_b, att_W, att_b, relatt_W, relatt_b, pred_W, pred_b):
    Xr = X.reshape(M, L * FIN)

    w_cur = att_W[:, :U, 0]
    w_nf = att_W[:, U:2 * U, 0]
    w_re = att_W[:, 2 * U:, 0]
    const_r = jnp.sum(rel_emb * w_re, axis=1) + att_b[:, 0]
    Wall = jnp.transpose(proj_W, (1, 0, 2)).reshape(U, R * U)
    ball = proj_b.reshape(1, R * U)
    eye = jnp.eye(R, dtype=jnp.float32)
    BD = jnp.concatenate(
        [(w_nf[:, :, None] * eye[:, None, :]).reshape(R * U, R),
         jnp.zeros((R * U, R), jnp.float32)], axis=1)
    Wc = jnp.concatenate(
        [jnp.zeros((U, R), jnp.float32), w_cur.T], axis=1)
    cvec = jnp.concatenate(
        [jnp.zeros((R,), jnp.float32), const_r]).reshape(1, 8)

    node, proj, scores = _run_stage_a(
        Xr, gru_kernel, gru_rec_kernel, gru_bias, Wall, ball, BD, Wc, cvec)

    s_nf_n = scores[:, :R]
    s_cur_n = scores[:, R:]
    proj0 = _leaky(proj_b)
    s_nf0 = jnp.sum(proj0 * w_nf, axis=1) - 1e9
    feat = proj.reshape(B, N, R, U).transpose(0, 2, 1, 3)
    row0 = jnp.broadcast_to(proj0[None, :, None, :], (B, R, 1, U))
    feat = jnp.concatenate([row0, feat], axis=2)
    snf = s_nf_n.reshape(B, N, R).transpose(0, 2, 1)
    snf = jnp.concatenate(
        [jnp.broadcast_to(s_nf0[None, :, None], (B, R, 1)), snf], axis=2)
    table = jnp.concatenate(
        [feat, snf[..., None],
         jnp.zeros((B, R, NP, WROW - U - 1), jnp.float32)],
        axis=3).reshape(P * NP, WROW)
    ctab = jnp.concatenate(
        [s_cur_n.reshape(B, N, R).transpose(0, 2, 1).reshape(ND),
         jnp.zeros((ND_PAD - ND,), jnp.float32)])
    pair_off = (jnp.arange(P, dtype=jnp.int32) * NP)[:, None]
    gidx = (neighbors.reshape(P, N * K).astype(jnp.int32) + pair_off)
    gidx = jnp.concatenate(
        [gidx.reshape(ND * K),
         jnp.zeros(((ND_PAD - ND) * K,), jnp.int32)])

    rep = _run_stage_c(table, gidx, ctab)[:ND]

    rep_all = rep.reshape(B, R, N, U).transpose(0, 2, 1, 3).reshape(M, R * U)

    w2_cur = relatt_W[:U, 0]
    w2_rep = relatt_W[U:2 * U, 0]
    w2_re = relatt_W[2 * U:, 0]
    const2 = rel_emb @ w2_re + relatt_b[0]
    bd2 = (jnp.broadcast_to(w2_rep[None, :, None], (R, U, R))
           * eye[:, None, :]).reshape(R * U, R)
    wc2 = jnp.broadcast_to(w2_cur[:, None], (U, R))
    c2 = const2.reshape(1, R)

    out = _run_stage_d(node, rep_all, bd2, wc2, c2, pred_W,
                       pred_b.reshape(1, NLAB))
    return out.reshape(B, N, NLAB)


# speedup vs baseline: 5.1025x; 1.5345x over previous
import functools
import jax
import jax.numpy as jnp
from jax import lax
from jax.experimental import pallas as pl
from jax.experimental.pallas import tpu as pltpu
from jax.experimental.pallas import tpu_sc as plsc

B, N, L, FIN = 4, 2000, 50, 8
U, R, K, NLAB, RD = 64, 4, 20, 3, 32
M = B * N
NP = N + 1
P = B * R
WROW = 128
ND = P * N
CH = 16
NCHUNK = ND // CH
NWORK = 32
EPB = CH * K
MB = 1000
GRID_A = M // MB


def _leaky(x):
    return jnp.where(x < 0, 0.2 * x, x)


def _stage_a_body(x_ref, gk_ref, grk_ref, gb_ref, wall_ref, ball_ref,
                  bd_ref, wc_ref, cv_ref, node_ref, proj_ref, sc_ref):
    x = x_ref[...]
    gk = gk_ref[...]
    grk = grk_ref[...]
    b0 = gb_ref[0:1, :]
    rb = gb_ref[1:2, :]
    h = jnp.zeros((MB, U), jnp.float32)
    for t in range(L):
        xt = x[:, t * FIN:(t + 1) * FIN]
        pre = jnp.dot(xt, gk, preferred_element_type=jnp.float32) + b0
        inner = jnp.dot(h, grk, preferred_element_type=jnp.float32) + rb
        z = jax.nn.sigmoid(pre[:, :U] + inner[:, :U])
        r = jax.nn.sigmoid(pre[:, U:2 * U] + inner[:, U:2 * U])
        hh = jnp.tanh(pre[:, 2 * U:] + r * inner[:, 2 * U:])
        h = z * h + (1.0 - z) * hh
    node_ref[...] = h
    pa = _leaky(jnp.dot(h, wall_ref[...], preferred_element_type=jnp.float32)
                + ball_ref[...])
    proj_ref[...] = pa
    sc_ref[...] = (jnp.dot(pa, bd_ref[...], preferred_element_type=jnp.float32)
                   + jnp.dot(h, wc_ref[...], preferred_element_type=jnp.float32)
                   + cv_ref[...])


def _run_stage_a(Xr, gk, grk, gb, Wall, ball, BD, Wc, cvec):
    full = lambda s: pl.BlockSpec(s, lambda i: (0, 0))
    return pl.pallas_call(
        _stage_a_body,
        grid=(GRID_A,),
        in_specs=[
            pl.BlockSpec((MB, L * FIN), lambda i: (i, 0)),
            full((FIN, 3 * U)), full((U, 3 * U)), full((2, 3 * U)),
            full((U, R * U)), full((1, R * U)),
            full((R * U, 8)), full((U, 8)), full((1, 8)),
        ],
        out_specs=[
            pl.BlockSpec((MB, U), lambda i: (i, 0)),
            pl.BlockSpec((MB, R * U), lambda i: (i, 0)),
            pl.BlockSpec((MB, 8), lambda i: (i, 0)),
        ],
        out_shape=[
            jax.ShapeDtypeStruct((M, U), jnp.float32),
            jax.ShapeDtypeStruct((M, R * U), jnp.float32),
            jax.ShapeDtypeStruct((M, 8), jnp.float32),
        ],
    )(Xr, gk, grk, gb, Wall, ball, BD, Wc, cvec)


NCHUNK_PAD = 2016
ND_PAD = NCHUNK_PAD * CH


def _stage_c_kernel(table_hbm, gidx_hbm, ctab_hbm, colmap_hbm, out_hbm,
                    gidx_v, rows_v, cv_v, colmap_v, w_v, acc_v, sem):
    nc = 2
    wid = lax.axis_index("s") * nc + lax.axis_index("c")
    pltpu.sync_copy(colmap_hbm, colmap_v)

    def chunk_body(j, carry):
        c = wid + NWORK * j
        dstart = c * CH
        pltpu.sync_copy(gidx_hbm.at[pl.ds(c * EPB, EPB)], gidx_v)
        pltpu.sync_copy(ctab_hbm.at[pl.ds(dstart, CH)], cv_v)

        cps = [pltpu.async_copy(table_hbm.at[gidx_v.at[pl.ds(q * 80, 80)]],
                                rows_v.at[pl.ds(q * 80, 80)], sem)
               for q in range(4)]
        for cp in cps:
            cp.wait()

        cv = cv_v[...]
        col = jnp.full((16,), U, jnp.int32)
        mx = jnp.full((16,), -1e30, jnp.float32)
        for k in range(K):
            rid = lax.iota(jnp.int32, 16) * K + k
            t = cv + plsc.load_gather(rows_v, [rid, col])
            t = jnp.where(t < 0.0, 0.2 * t, t)
            w_v[pl.ds(k * CH, CH)] = t
            mx = jnp.maximum(mx, t)
        den = jnp.zeros((16,), jnp.float32)
        for k in range(K):
            e = jnp.exp(w_v[pl.ds(k * CH, CH)] - mx)
            w_v[pl.ds(k * CH, CH)] = e
            den = den + e
        inv = 1.0 / den

        wks = [w_v[pl.ds(k * CH, CH)] * inv for k in range(K)]
        for i in range(CH):
            base = i * K
            ws = [wks[k][i] for k in range(K)]
            for v4 in range(4):
                acc = jnp.zeros((16,), jnp.float32)
                for k in range(K):
                    acc = acc + ws[k] * rows_v[base + k, pl.ds(v4 * 16, 16)]
                acc_v[i, pl.ds(v4 * 16, 16)] = acc
        pltpu.sync_copy(acc_v, out_hbm.at[pl.ds(dstart, CH)])
        return carry

    lax.fori_loop(0, NCHUNK_PAD // NWORK, chunk_body, 0)


def _run_stage_c(table, gidx, ctab):
    colmap = jnp.broadcast_to(
        jnp.arange(U, dtype=jnp.int32)[:, None], (U, CH)).reshape(U * CH)
    mesh = plsc.VectorSubcoreMesh(core_axis_name="c", subcore_axis_name="s")
    kfn = functools.partial(
        pl.kernel,
        mesh=mesh,
        compiler_params=pltpu.CompilerParams(needs_layout_passes=False),
        out_type=jax.ShapeDtypeStruct((ND_PAD, U), jnp.float32),
        scratch_types=[
            pltpu.VMEM((EPB,), jnp.int32),
            pltpu.VMEM((EPB, WROW), jnp.float32),
            pltpu.VMEM((CH,), jnp.float32),
            pltpu.VMEM((U * CH,), jnp.int32),
            pltpu.VMEM((K * CH,), jnp.float32),
            pltpu.VMEM((CH, U), jnp.float32),
            pltpu.SemaphoreType.DMA,
        ],
    )(_stage_c_kernel)
    return kfn(table, gidx, ctab, colmap)


def _stage_d_body(node_ref, rep_ref, bd2_ref, wc2_ref, c2_ref,
                  pw_ref, pb_ref, out_ref):
    h = node_ref[...]
    rep = rep_ref[...]
    sc2 = (jnp.dot(rep, bd2_ref[...], preferred_element_type=jnp.float32)
           + jnp.dot(h, wc2_ref[...], preferred_element_type=jnp.float32)
           + c2_ref[...])
    sc2 = _leaky(sc2)
    m2 = jnp.max(sc2, axis=1, keepdims=True)
    e2 = jnp.exp(sc2 - m2)
    w2 = e2 / jnp.sum(e2, axis=1, keepdims=True)
    upd = h
    for r in range(R):
        upd = upd + rep[:, r * U:(r + 1) * U] * w2[:, r:r + 1]
    logits = (jnp.dot(upd, pw_ref[...], preferred_element_type=jnp.float32)
              + pb_ref[...])
    ml = jnp.max(logits, axis=1, keepdims=True)
    el = jnp.exp(logits - ml)
    out_ref[...] = el / jnp.sum(el, axis=1, keepdims=True)


def _run_stage_d(node, rep_all, bd2, wc2, c2, pw, pb):
    full = lambda s: pl.BlockSpec(s, lambda i: (0, 0))
    return pl.pallas_call(
        _stage_d_body,
        grid=(GRID_A,),
        in_specs=[
            pl.BlockSpec((MB, U), lambda i: (i, 0)),
            pl.BlockSpec((MB, R * U), lambda i: (i, 0)),
            full((R * U, R)), full((U, R)), full((1, R)),
            full((U, NLAB)), full((1, NLAB)),
        ],
        out_specs=pl.BlockSpec((MB, NLAB), lambda i: (i, 0)),
        out_shape=jax.ShapeDtypeStruct((M, NLAB), jnp.float32),
    )(node, rep_all, bd2, wc2, c2, pw, pb)


def kernel(X, neighbors, gru_kernel, gru_rec_kernel, gru_bias, rel_emb,
           proj_W, proj_b, att_W, att_b, relatt_W, relatt_b, pred_W, pred_b):
    Xr = X.reshape(M, L * FIN)

    w_cur = att_W[:, :U, 0]
    w_nf = att_W[:, U:2 * U, 0]
    w_re = att_W[:, 2 * U:, 0]
    const_r = jnp.sum(rel_emb * w_re, axis=1) + att_b[:, 0]
    Wall = jnp.transpose(proj_W, (1, 0, 2)).reshape(U, R * U)
    ball = proj_b.reshape(1, R * U)
    eye = jnp.eye(R, dtype=jnp.float32)
    BD = jnp.concatenate(
        [(w_nf[:, :, None] * eye[:, None, :]).reshape(R * U, R),
         jnp.zeros((R * U, R), jnp.float32)], axis=1)
    Wc = jnp.concatenate(
        [jnp.zeros((U, R), jnp.float32), w_cur.T], axis=1)
    cvec = jnp.concatenate(
        [jnp.zeros((R,), jnp.float32), const_r]).reshape(1, 8)

    node, proj, scores = _run_stage_a(
        Xr, gru_kernel, gru_rec_kernel, gru_bias, Wall, ball, BD, Wc, cvec)

    s_nf_n = scores[:, :R]
    s_cur_n = scores[:, R:]
    proj0 = _leaky(proj_b)
    s_nf0 = jnp.sum(proj0 * w_nf, axis=1) - 1e9
    feat = proj.reshape(B, N, R, U).transpose(0, 2, 1, 3)
    row0 = jnp.broadcast_to(proj0[None, :, None, :], (B, R, 1, U))
    feat = jnp.concatenate([row0, feat], axis=2)
    snf = s_nf_n.reshape(B, N, R).transpose(0, 2, 1)
    snf = jnp.concatenate(
        [jnp.broadcast_to(s_nf0[None, :, None], (B, R, 1)), snf], axis=2)
    table = jnp.concatenate(
        [feat, snf[..., None],
         jnp.zeros((B, R, NP, WROW - U - 1), jnp.float32)],
        axis=3).reshape(P * NP, WROW)
    ctab = jnp.concatenate(
        [s_cur_n.reshape(B, N, R).transpose(0, 2, 1).reshape(ND),
         jnp.zeros((ND_PAD - ND,), jnp.float32)])
    pair_off = (jnp.arange(P, dtype=jnp.int32) * NP)[:, None]
    gidx = (neighbors.reshape(P, N * K).astype(jnp.int32) + pair_off)
    gidx = jnp.concatenate(
        [gidx.reshape(ND * K),
         jnp.zeros(((ND_PAD - ND) * K,), jnp.int32)])

    rep = _run_stage_c(table, gidx, ctab)[:ND]

    rep_all = rep.reshape(B, R, N, U).transpose(0, 2, 1, 3).reshape(M, R * U)

    w2_cur = relatt_W[:U, 0]
    w2_rep = relatt_W[U:2 * U, 0]
    w2_re = relatt_W[2 * U:, 0]
    const2 = rel_emb @ w2_re + relatt_b[0]
    bd2 = (jnp.broadcast_to(w2_rep[None, :, None], (R, U, R))
           * eye[:, None, :]).reshape(R * U, R)
    wc2 = jnp.broadcast_to(w2_cur[:, None], (U, R))
    c2 = const2.reshape(1, R)

    out = _run_stage_d(node, rep_all, bd2, wc2, c2, pred_W,
                       pred_b.reshape(1, NLAB))
    return out.reshape(B, N, NLAB)
